# Initial kernel scaffold; baseline (speedup 1.0000x reference)
import jax
import jax.numpy as jnp
from jax.experimental import pallas as pl


def kernel(edge_index2, h, t_ij, Wq, bq, Wk, bk, Wv, bv, Wg, bg, Wo, bo, We1, be1, We2, be2):
    raise NotImplementedError("write your pallas kernel here")


# speedup vs baseline: 2.0542x; 2.0542x over previous
import functools

import jax
import jax.numpy as jnp
from jax.experimental import pallas as pl
from jax.experimental.pallas import tpu as pltpu
from jax.experimental.pallas import tpu_sc as plsc

N = 10000
E = 160000
D = 128
H = 8
DH = D // H

NC = 2
NS = 16
NW = NC * NS

BN = 2000
BE = 2000

_f32 = jnp.float32
_bf16 = jnp.bfloat16


def _mm(a, w):
    return jax.lax.dot(a.astype(_bf16), w.astype(_bf16),
                       preferred_element_type=_f32)


def _head_matrix(dtype):
    d = jax.lax.broadcasted_iota(jnp.int32, (D, H), 0)
    h = jax.lax.broadcasted_iota(jnp.int32, (D, H), 1)
    return ((d // DH) == h).astype(dtype)


def _sc_mesh():
    return plsc.VectorSubcoreMesh(core_axis_name="c", subcore_axis_name="s",
                                  num_cores=NC, num_subcores=NS)


def _sc_gather_rows(table, idx, chunk):
    m = idx.shape[0]
    d = table.shape[1]
    per_w = m // NW
    n_chunks = per_w // chunk

    @functools.partial(
        pl.kernel,
        out_type=jax.ShapeDtypeStruct((m, d), table.dtype),
        mesh=_sc_mesh(),
        scratch_types=[pltpu.VMEM((chunk,), jnp.int32),
                       pltpu.VMEM((chunk, d), table.dtype)],
    )
    def k(tab_hbm, idx_hbm, out_hbm, idx_v, rows_v):
        wid = jax.lax.axis_index("s") * NC + jax.lax.axis_index("c")

        @pl.loop(0, n_chunks)
        def _(i):
            base = wid * per_w + i * chunk
            pltpu.sync_copy(idx_hbm.at[pl.ds(base, chunk)], idx_v)
            pltpu.sync_copy(tab_hbm.at[idx_v], rows_v)
            pltpu.sync_copy(rows_v, out_hbm.at[pl.ds(base, chunk)])

    return k(table, idx)


def _sc_scatter_add(msg, dst, zeros, chunk):
    per_w = E // NW
    n_chunks = per_w // chunk
    rows_per_init = N // 10

    @functools.partial(
        pl.kernel,
        out_type=jax.ShapeDtypeStruct((NC * N, D), _f32),
        mesh=_sc_mesh(),
        scratch_types=[pltpu.VMEM((chunk,), jnp.int32),
                       pltpu.VMEM((chunk, D), _f32),
                       pltpu.VMEM_SHARED((N, D), _f32)],
    )
    def k(msg_hbm, dst_hbm, z_hbm, u_hbm, idx_v, rows_v, acc_sh):
        cid = jax.lax.axis_index("c")
        sid = jax.lax.axis_index("s")
        wid = sid * NC + cid

        @pl.when(sid < 10)
        def _():
            sl = pl.ds(sid * rows_per_init, rows_per_init)
            pltpu.sync_copy(z_hbm.at[sl], acc_sh.at[sl])

        plsc.subcore_barrier()

        @pl.loop(0, n_chunks)
        def _(i):
            base = wid * per_w + i * chunk
            pltpu.sync_copy(dst_hbm.at[pl.ds(base, chunk)], idx_v)
            pltpu.sync_copy(msg_hbm.at[pl.ds(base, chunk)], rows_v)
            pltpu.sync_copy(rows_v, acc_sh.at[idx_v], add=True)

        plsc.subcore_barrier()

        @pl.when(sid < 10)
        def _():
            sl = pl.ds(sid * rows_per_init, rows_per_init)
            pltpu.sync_copy(acc_sh.at[sl],
                            u_hbm.at[pl.ds(cid * N + sid * rows_per_init,
                                           rows_per_init)])

    return k(msg, dst, zeros)


def _k1_qkv(h, Wstack, bstack):
    def body(h_ref, w_ref, b_ref, t_ref):
        t_ref[...] = _mm(h_ref[...], w_ref[0]) + b_ref[0]

    return pl.pallas_call(
        body,
        grid=(3, N // BN),
        in_specs=[
            pl.BlockSpec((BN, D), lambda w, i: (i, 0)),
            pl.BlockSpec((1, D, D), lambda w, i: (w, 0, 0)),
            pl.BlockSpec((1, 1, D), lambda w, i: (w, 0, 0)),
        ],
        out_specs=pl.BlockSpec((BN, D), lambda w, i: (w * (N // BN) + i, 0)),
        out_shape=jax.ShapeDtypeStruct((3 * N, D), _f32),
    )(h, Wstack, bstack)


def _k3_logits(gqk, t_ij, Wg, bg):
    def body(g_ref, t_ref, wg_ref, bg_ref, l_ref, m_ref, z_ref):
        qd = g_ref[:, :D]
        ks = g_ref[:, D:]
        mhead = _head_matrix(_bf16)
        prod = jax.lax.dot((qd * ks).astype(_bf16), mhead,
                           preferred_element_type=_f32) * (1.0 / (DH ** 0.5))
        logit = prod + _mm(t_ref[...], wg_ref[...]) + bg_ref[...]
        l_ref[...] = logit

        @pl.when(pl.program_id(1) == 0)
        def _():
            m_ref[...] = jnp.full((1, H), -jnp.inf, _f32)
            z_ref[...] = jnp.zeros((1, H), _f32)

        bm = jnp.max(logit, axis=0, keepdims=True)
        m_old = m_ref[...]
        m_new = jnp.maximum(m_old, bm)
        z_ref[...] = (z_ref[...] * jnp.exp(m_old - m_new)
                      + jnp.sum(jnp.exp(logit - m_new), axis=0, keepdims=True))
        m_ref[...] = m_new

    return pl.pallas_call(
        body,
        grid=(1, E // BE),
        in_specs=[
            pl.BlockSpec((BE, 2 * D), lambda _, i: (i, 0)),
            pl.BlockSpec((BE, D), lambda _, i: (i, 0)),
            pl.BlockSpec((D, H), lambda _, i: (0, 0)),
            pl.BlockSpec((1, H), lambda _, i: (0, 0)),
        ],
        out_specs=[
            pl.BlockSpec((BE, H), lambda _, i: (i, 0)),
            pl.BlockSpec((1, H), lambda _, i: (0, 0)),
            pl.BlockSpec((1, H), lambda _, i: (0, 0)),
        ],
        out_shape=[
            jax.ShapeDtypeStruct((E, H), _f32),
            jax.ShapeDtypeStruct((1, H), _f32),
            jax.ShapeDtypeStruct((1, H), _f32),
        ],
    )(gqk, t_ij, Wg, bg)


def _k5_msg(l, m, vs):
    def body(l_ref, m_ref, v_ref, msg_ref):
        p = jnp.exp(l_ref[...] - m_ref[...])
        mheadT = _head_matrix(_bf16).T
        p128 = jax.lax.dot(p.astype(_bf16), mheadT,
                           preferred_element_type=_f32)
        msg_ref[...] = p128 * v_ref[...]

    return pl.pallas_call(
        body,
        grid=(E // BE,),
        in_specs=[
            pl.BlockSpec((BE, H), lambda i: (i, 0)),
            pl.BlockSpec((1, H), lambda i: (0, 0)),
            pl.BlockSpec((BE, D), lambda i: (i, 0)),
        ],
        out_specs=pl.BlockSpec((BE, D), lambda i: (i, 0)),
        out_shape=jax.ShapeDtypeStruct((E, D), _f32),
    )(l, m, vs)


def _k7a_hnew(u, z, h, Wo, bo):
    def body(u0_ref, u1_ref, z_ref, h_ref, wo_ref, bo_ref, o_ref):
        mheadT = _head_matrix(_bf16).T
        r = jax.lax.dot((1.0 / z_ref[...]).astype(_bf16), mheadT,
                        preferred_element_type=_f32)
        un = (u0_ref[...] + u1_ref[...]) * r
        o_ref[...] = h_ref[...] + _mm(un, wo_ref[...]) + bo_ref[...]

    nb = N // BN
    return pl.pallas_call(
        body,
        grid=(nb,),
        in_specs=[
            pl.BlockSpec((BN, D), lambda i: (i, 0)),
            pl.BlockSpec((BN, D), lambda i: (i + nb, 0)),
            pl.BlockSpec((1, H), lambda i: (0, 0)),
            pl.BlockSpec((BN, D), lambda i: (i, 0)),
            pl.BlockSpec((D, D), lambda i: (0, 0)),
            pl.BlockSpec((1, D), lambda i: (0, 0)),
        ],
        out_specs=pl.BlockSpec((BN, D), lambda i: (i, 0)),
        out_shape=jax.ShapeDtypeStruct((N, D), _f32),
    )(u, u, z, h, Wo, bo)


def _k7b_ab(h_new, We1ab):
    def body(h_ref, w_ref, t_ref):
        t_ref[...] = _mm(h_ref[...], w_ref[0])

    return pl.pallas_call(
        body,
        grid=(2, N // BN),
        in_specs=[
            pl.BlockSpec((BN, D), lambda w, i: (i, 0)),
            pl.BlockSpec((1, D, D), lambda w, i: (w, 0, 0)),
        ],
        out_specs=pl.BlockSpec((BN, D), lambda w, i: (w * (N // BN) + i, 0)),
        out_shape=jax.ShapeDtypeStruct((2 * N, D), _f32),
    )(h_new, We1ab)


def _k9_tnew(gab, t_ij, We1c, be1, We2, be2):
    def body(g_ref, t_ref, w1_ref, b1_ref, w2_ref, b2_ref, o_ref):
        s = g_ref[:, :D] + g_ref[:, D:]
        pre = s + _mm(t_ref[...], w1_ref[...]) + b1_ref[...]
        act = pre * jax.nn.sigmoid(pre)
        o_ref[...] = t_ref[...] + _mm(act, w2_ref[...]) + b2_ref[...]

    return pl.pallas_call(
        body,
        grid=(E // BE,),
        in_specs=[
            pl.BlockSpec((BE, 2 * D), lambda i: (i, 0)),
            pl.BlockSpec((BE, D), lambda i: (i, 0)),
            pl.BlockSpec((D, D), lambda i: (0, 0)),
            pl.BlockSpec((1, D), lambda i: (0, 0)),
            pl.BlockSpec((D, D), lambda i: (0, 0)),
            pl.BlockSpec((1, D), lambda i: (0, 0)),
        ],
        out_specs=pl.BlockSpec((BE, D), lambda i: (i, 0)),
        out_shape=jax.ShapeDtypeStruct((E, D), _f32),
    )(gab, t_ij, We1c, be1, We2, be2)


def kernel(edge_index2, h, t_ij, Wq, bq, Wk, bk, Wv, bv, Wg, bg, Wo, bo,
           We1, be1, We2, be2):
    src = edge_index2[0]
    dst = edge_index2[1]

    Wstack = jnp.stack([Wq, Wk, Wv])
    bstack = jnp.stack([bq, bk, bv]).reshape(3, 1, D)
    T = _k1_qkv(h, Wstack, bstack)

    iqk = jnp.stack([dst, src + N], axis=1).reshape(-1)
    gqk = _sc_gather_rows(T, iqk, chunk=1000).reshape(E, 2 * D)
    vs = _sc_gather_rows(T, src + 2 * N, chunk=1000)

    l, m, z = _k3_logits(gqk, t_ij, Wg, bg.reshape(1, H))
    msg = _k5_msg(l, m, vs)

    zeros = jnp.zeros((N, D), _f32)
    u = _sc_scatter_add(msg, dst, zeros, chunk=200)

    h_new = _k7a_hnew(u, z, h, Wo, bo.reshape(1, D))
    tab = _k7b_ab(h_new, jnp.stack([We1[:D], We1[D:2 * D]]))

    iab = jnp.stack([src, dst + N], axis=1).reshape(-1)
    gab = _sc_gather_rows(tab, iab, chunk=1000).reshape(E, 2 * D)

    t_new = _k9_tnew(gab, t_ij, We1[2 * D:], be1.reshape(1, D),
                     We2, be2.reshape(1, D))
    return (h_new, t_new)
